# baseline (speedup 1.0000x reference)
import jax
import jax.numpy as jnp
from jax import lax
from jax.experimental import pallas as pl
from jax.experimental.pallas import tpu as pltpu
from jax.experimental.pallas import tpu_sc as plsc

N = 10000
D_IN = 128
CHANNELS = 128
N_LABELS = 64

NC = 2
NS = 16
NW = NC * NS
ROW_BLK = 2000
GRID = N // ROW_BLK

NSLOT = 4


def _sc_aggregate(C, E, CH):
  per_w = -(-E // NW)
  K = -(-per_w // CH)
  assert K >= 8
  assert per_w % 8 == 0 and E % 8 == 0
  rows_per_s = N // NS
  nb = C // 16
  mesh = plsc.VectorSubcoreMesh(core_axis_name="c", subcore_axis_name="s",
                                num_cores=NC, num_subcores=NS)

  def body(h_hbm, src_hbm, dst_hbm, w_hbm, out_hbm, *scr):
    srcb = scr[0:NSLOT]
    dstb = scr[NSLOT:2 * NSLOT]
    wbv = scr[2 * NSLOT:3 * NSLOT]
    rows = scr[3 * NSLOT:4 * NSLOT]
    acc = scr[4 * NSLOT]
    sw_sem = scr[4 * NSLOT + 1:4 * NSLOT + 1 + NSLOT]
    d_sem = scr[4 * NSLOT + 1 + NSLOT:4 * NSLOT + 1 + 2 * NSLOT]
    g_sem = scr[4 * NSLOT + 1 + 2 * NSLOT:4 * NSLOT + 1 + 3 * NSLOT]
    s_sem = scr[4 * NSLOT + 1 + 3 * NSLOT:4 * NSLOT + 1 + 4 * NSLOT]

    c = lax.axis_index("c")
    s = lax.axis_index("s")
    wid = s * NC + c
    base = wid * per_w
    vcnt = jnp.clip(E - base, 0, per_w)
    maxoff = jnp.maximum(vcnt - CH, 0)

    def off_of(g):
      if isinstance(g, int) and g < K - 1:
        return base + g * CH
      return pl.multiple_of(base + jnp.minimum(g * CH, maxoff), 8)

    def issue_sw(g, u):
      off = off_of(g)
      pltpu.async_copy(src_hbm.at[pl.ds(off, CH)], srcb[u], sw_sem[u])
      pltpu.async_copy(w_hbm.at[pl.ds(off, CH)], wbv[u], sw_sem[u])

    def wait_sw(u):
      pltpu.make_async_copy(src_hbm.at[pl.ds(0, CH)], srcb[u],
                            sw_sem[u]).wait()
      pltpu.make_async_copy(w_hbm.at[pl.ds(0, CH)], wbv[u],
                            sw_sem[u]).wait()

    def issue_dst(g, u):
      pltpu.async_copy(dst_hbm.at[pl.ds(off_of(g), CH)], dstb[u], d_sem[u])

    def wait_dst(u):
      pltpu.make_async_copy(dst_hbm.at[pl.ds(0, CH)], dstb[u],
                            d_sem[u]).wait()

    def issue_gather(u):
      pltpu.async_copy(h_hbm.at[srcb[u]], rows[u], g_sem[u])

    def wait_gather(u):
      pltpu.make_async_copy(h_hbm.at[srcb[u]], rows[u], g_sem[u]).wait()

    def issue_scatter(u):
      pltpu.async_copy(rows[u], acc.at[dstb[u]], s_sem[u], add=True)

    def wait_scatter(u):
      pltpu.make_async_copy(rows[u], acc.at[dstb[u]], s_sem[u]).wait()

    def scale_grp(rbuf, wvm, e0):
      for i in range(16):
        ws = lax.gather(
            wvm, jnp.full((16, 1), i, jnp.int32),
            dimension_numbers=lax.GatherDimensionNumbers(
                offset_dims=(), collapsed_slice_dims=(0,),
                start_index_map=(0,)),
            slice_sizes=(1,),
            mode=lax.GatherScatterMode.PROMISE_IN_BOUNDS)
        for j in range(nb):
          sl = pl.ds(16 * j, 16)
          rbuf[e0 + i, sl] = rbuf[e0 + i, sl] * ws

    def scale_full(u):
      rbuf = rows[u]
      wbu = wbv[u]

      def grp(k, _):
        e0 = k * 16
        scale_grp(rbuf, wbu[pl.ds(e0, 16)], e0)
        return 0

      lax.fori_loop(0, CH // 16, grp, 0)

    def scale_masked(u):
      rbuf = rows[u]
      wbu = wbv[u]
      off = off_of(K - 1)
      lo = base + (K - 1) * CH
      hi = base + vcnt

      def grp(k, _):
        e0 = k * 16
        ids = jnp.arange(16, dtype=jnp.int32) + (off + e0)
        wv = wbu[pl.ds(e0, 16)]
        wvm = jnp.where((ids >= lo) & (ids < hi), wv, jnp.float32(0.0))
        scale_grp(rbuf, wvm, e0)
        return 0

      lax.fori_loop(0, CH // 16, grp, 0)

    def pre(g, u):
      wait_sw(u)
      issue_dst(g, u)
      issue_gather(u)

    def post(u, refill, masked=False):
      wait_gather(u)
      if masked:
        scale_masked(u)
      else:
        scale_full(u)
      wait_dst(u)
      issue_scatter(u)
      if isinstance(refill, int):
        if refill < K:
          issue_sw(refill, u)
      else:
        @pl.when(refill < K)
        def _():
          issue_sw(refill, u)

    for u in range(NSLOT):
      issue_sw(u, u)

    def zrow(i, _):
      for j in range(nb):
        rows[0][i, pl.ds(16 * j, 16)] = jnp.zeros((16,), jnp.float32)
      return 0
    lax.fori_loop(0, CH, zrow, 0)
    r0 = s * rows_per_s
    zoff = 0
    while zoff < rows_per_s:
      n = min(CH, rows_per_s - zoff)
      pltpu.sync_copy(rows[0].at[pl.ds(0, n)], acc.at[pl.ds(r0 + zoff, n)])
      zoff += n
    plsc.subcore_barrier()

    pre(0, 0)
    pre(1, 1)
    post(0, 4)
    pre(2, 2)
    post(1, 5)
    pre(3, 3)
    post(2, 6)

    G = 4 * ((K - 1) // 4)

    def steady(t, _):
      g0 = 4 * t
      for u in range(NSLOT):
        g = g0 + u
        wait_scatter(u)
        pre(g, u)
        post((u + 3) % 4, g + 3)
      return 0

    lax.fori_loop(1, G // 4, steady, 0)

    for g in range(G, K - 1):
      wait_scatter(g % 4)
      pre(g, g % 4)
      post((g - 1) % 4, g + 3)
    uK = (K - 1) % 4
    wait_scatter(uK)
    pre(K - 1, uK)
    post((K - 2) % 4, K + 4)
    post(uK, K + 4, masked=True)
    for u in range(NSLOT):
      wait_scatter(u)

    plsc.subcore_barrier()
    pltpu.sync_copy(acc.at[pl.ds(r0, rows_per_s)],
                    out_hbm.at[c, pl.ds(r0, rows_per_s)])

  return pl.kernel(
      body,
      out_type=jax.ShapeDtypeStruct((NC, N, C), jnp.float32),
      mesh=mesh,
      compiler_params=pltpu.CompilerParams(use_tc_tiling_on_sc=False,
                                           needs_layout_passes=False),
      scratch_types=(
          [pltpu.VMEM((CH,), jnp.int32) for _ in range(NSLOT)]
          + [pltpu.VMEM((CH,), jnp.int32) for _ in range(NSLOT)]
          + [pltpu.VMEM((CH,), jnp.float32) for _ in range(NSLOT)]
          + [pltpu.VMEM((CH, C), jnp.float32) for _ in range(NSLOT)]
          + [pltpu.VMEM_SHARED((N, C), jnp.float32)]
          + [pltpu.SemaphoreType.DMA for _ in range(4 * NSLOT)]
      ),
  )


def _tc_matmul(x, w):
  K, C = w.shape

  def body(x_ref, w_ref, o_ref):
    o_ref[...] = jnp.dot(x_ref[...], w_ref[...],
                         preferred_element_type=jnp.float32)

  return pl.pallas_call(
      body,
      grid=(GRID,),
      in_specs=[pl.BlockSpec((ROW_BLK, K), lambda i: (i, 0)),
                pl.BlockSpec((K, C), lambda i: (0, 0))],
      out_specs=pl.BlockSpec((ROW_BLK, C), lambda i: (i, 0)),
      out_shape=jax.ShapeDtypeStruct((N, C), jnp.float32),
  )(x, w)


def _tc_add_relu_matmul2(p2, w2):
  def body(p_ref, w_ref, o_ref):
    h = jnp.maximum(p_ref[0] + p_ref[1], 0.0)
    o_ref[...] = jnp.dot(h, w_ref[...], preferred_element_type=jnp.float32)

  return pl.pallas_call(
      body,
      grid=(GRID,),
      in_specs=[pl.BlockSpec((NC, ROW_BLK // 2, 256), lambda i: (0, i, 0)),
                pl.BlockSpec((256, 128), lambda i: (0, 0))],
      out_specs=pl.BlockSpec((ROW_BLK // 2, 128), lambda i: (i, 0)),
      out_shape=jax.ShapeDtypeStruct((N // 2, 128), jnp.float32),
  )(p2, w2)


def _tc_add_softmax2(q2):
  def body(q_ref, o_ref):
    z = q_ref[0] + q_ref[1]

    def sm(zz):
      m = jnp.max(zz, axis=-1, keepdims=True)
      e = jnp.exp(zz - m)
      return e / jnp.sum(e, axis=-1, keepdims=True)

    o_ref[:, pl.ds(0, 64)] = sm(z[:, :64])
    o_ref[:, pl.ds(64, 64)] = sm(z[:, 64:])

  return pl.pallas_call(
      body,
      grid=(GRID,),
      in_specs=[pl.BlockSpec((NC, ROW_BLK // 2, 128), lambda i: (0, i, 0))],
      out_specs=pl.BlockSpec((ROW_BLK // 2, 128), lambda i: (i, 0)),
      out_shape=jax.ShapeDtypeStruct((N // 2, 128), jnp.float32),
  )(q2)


def kernel(x, edge_index, edge_weight, W0, W1):
  E = edge_weight.shape[0]
  src = edge_index[0]
  dst = edge_index[1]
  w2 = jnp.zeros((256, 128), jnp.float32)
  w2 = w2.at[:128, :64].set(W1).at[128:, 64:].set(W1)
  h0 = _tc_matmul(x, W0)
  p = _sc_aggregate(CHANNELS, E, 64)(h0, src, dst, edge_weight)
  p2 = jnp.reshape(p, (NC, N // 2, 256))
  h1 = jnp.reshape(_tc_add_relu_matmul2(p2, w2), (N, N_LABELS))
  q = _sc_aggregate(N_LABELS, E, 128)(h1, src, dst, edge_weight)
  q2 = jnp.reshape(q, (NC, N // 2, 2 * N_LABELS))
  return jnp.reshape(_tc_add_softmax2(q2), (N, N_LABELS))


# speedup vs baseline: 1.0657x; 1.0657x over previous
import jax
import jax.numpy as jnp
from jax import lax
from jax.experimental import pallas as pl
from jax.experimental.pallas import tpu as pltpu
from jax.experimental.pallas import tpu_sc as plsc

N = 10000
D_IN = 128
CHANNELS = 128
N_LABELS = 64

NC = 2
NS = 16
NW = NC * NS
ROW_BLK = 2000
GRID = N // ROW_BLK

NSLOT = 4


def _sc_aggregate(C, E, CH):
  per_w = -(-E // NW)
  K = -(-per_w // CH)
  assert K >= 8
  assert per_w % 8 == 0 and E % 8 == 0
  rows_per_s = N // NS
  nb = C // 16
  mesh = plsc.VectorSubcoreMesh(core_axis_name="c", subcore_axis_name="s",
                                num_cores=NC, num_subcores=NS)

  def body(h_hbm, ei_hbm, w_hbm, out_hbm, *scr):
    srcb = scr[0:NSLOT]
    dstb = scr[NSLOT:2 * NSLOT]
    wbv = scr[2 * NSLOT:3 * NSLOT]
    rows = scr[3 * NSLOT:4 * NSLOT]
    acc = scr[4 * NSLOT]
    sw_sem = scr[4 * NSLOT + 1:4 * NSLOT + 1 + NSLOT]
    d_sem = scr[4 * NSLOT + 1 + NSLOT:4 * NSLOT + 1 + 2 * NSLOT]
    g_sem = scr[4 * NSLOT + 1 + 2 * NSLOT:4 * NSLOT + 1 + 3 * NSLOT]
    s_sem = scr[4 * NSLOT + 1 + 3 * NSLOT:4 * NSLOT + 1 + 4 * NSLOT]

    c = lax.axis_index("c")
    s = lax.axis_index("s")
    wid = s * NC + c
    base = wid * per_w
    vcnt = jnp.clip(E - base, 0, per_w)
    maxoff = jnp.maximum(vcnt - CH, 0)

    def off_of(g):
      if isinstance(g, int) and g < K - 1:
        return base + g * CH
      return pl.multiple_of(base + jnp.minimum(g * CH, maxoff), 8)

    def issue_sw(g, u):
      off = off_of(g)
      pltpu.async_copy(ei_hbm.at[0, pl.ds(off, CH)], srcb[u], sw_sem[u])
      pltpu.async_copy(w_hbm.at[pl.ds(off, CH)], wbv[u], sw_sem[u])

    def wait_sw(u):
      pltpu.make_async_copy(ei_hbm.at[0, pl.ds(0, CH)], srcb[u],
                            sw_sem[u]).wait()
      pltpu.make_async_copy(w_hbm.at[pl.ds(0, CH)], wbv[u],
                            sw_sem[u]).wait()

    def issue_dst(g, u):
      pltpu.async_copy(ei_hbm.at[1, pl.ds(off_of(g), CH)], dstb[u], d_sem[u])

    def wait_dst(u):
      pltpu.make_async_copy(ei_hbm.at[1, pl.ds(0, CH)], dstb[u],
                            d_sem[u]).wait()

    def issue_gather(u):
      pltpu.async_copy(h_hbm.at[srcb[u]], rows[u], g_sem[u])

    def wait_gather(u):
      pltpu.make_async_copy(h_hbm.at[srcb[u]], rows[u], g_sem[u]).wait()

    def issue_scatter(u):
      pltpu.async_copy(rows[u], acc.at[dstb[u]], s_sem[u], add=True)

    def wait_scatter(u):
      pltpu.make_async_copy(rows[u], acc.at[dstb[u]], s_sem[u]).wait()

    def scale_grp(rbuf, wvm, e0):
      for i in range(16):
        ws = lax.gather(
            wvm, jnp.full((16, 1), i, jnp.int32),
            dimension_numbers=lax.GatherDimensionNumbers(
                offset_dims=(), collapsed_slice_dims=(0,),
                start_index_map=(0,)),
            slice_sizes=(1,),
            mode=lax.GatherScatterMode.PROMISE_IN_BOUNDS)
        for j in range(nb):
          sl = pl.ds(16 * j, 16)
          rbuf[e0 + i, sl] = rbuf[e0 + i, sl] * ws

    def scale_full(u):
      rbuf = rows[u]
      wbu = wbv[u]

      def grp(k, _):
        e0 = k * 16
        scale_grp(rbuf, wbu[pl.ds(e0, 16)], e0)
        return 0

      lax.fori_loop(0, CH // 16, grp, 0)

    def scale_masked(u):
      rbuf = rows[u]
      wbu = wbv[u]
      off = off_of(K - 1)
      lo = base + (K - 1) * CH
      hi = base + vcnt

      def grp(k, _):
        e0 = k * 16
        ids = jnp.arange(16, dtype=jnp.int32) + (off + e0)
        wv = wbu[pl.ds(e0, 16)]
        wvm = jnp.where((ids >= lo) & (ids < hi), wv, jnp.float32(0.0))
        scale_grp(rbuf, wvm, e0)
        return 0

      lax.fori_loop(0, CH // 16, grp, 0)

    def pre(g, u):
      wait_sw(u)
      issue_dst(g, u)
      issue_gather(u)

    def post(u, refill, masked=False):
      wait_gather(u)
      if masked:
        scale_masked(u)
      else:
        scale_full(u)
      wait_dst(u)
      issue_scatter(u)
      if isinstance(refill, int):
        if refill < K:
          issue_sw(refill, u)
      else:
        @pl.when(refill < K)
        def _():
          issue_sw(refill, u)

    for u in range(NSLOT):
      issue_sw(u, u)

    def zrow(i, _):
      for j in range(nb):
        rows[0][i, pl.ds(16 * j, 16)] = jnp.zeros((16,), jnp.float32)
      return 0
    lax.fori_loop(0, CH, zrow, 0)
    r0 = s * rows_per_s
    zoff = 0
    while zoff < rows_per_s:
      n = min(CH, rows_per_s - zoff)
      pltpu.sync_copy(rows[0].at[pl.ds(0, n)], acc.at[pl.ds(r0 + zoff, n)])
      zoff += n
    plsc.subcore_barrier()

    pre(0, 0)
    pre(1, 1)
    post(0, 4)
    pre(2, 2)
    post(1, 5)
    pre(3, 3)
    post(2, 6)

    G = 4 * ((K - 1) // 4)

    def steady(t, _):
      g0 = 4 * t
      for u in range(NSLOT):
        g = g0 + u
        wait_scatter(u)
        pre(g, u)
        post((u + 3) % 4, g + 3)
      return 0

    lax.fori_loop(1, G // 4, steady, 0)

    for g in range(G, K - 1):
      wait_scatter(g % 4)
      pre(g, g % 4)
      post((g - 1) % 4, g + 3)
    uK = (K - 1) % 4
    wait_scatter(uK)
    pre(K - 1, uK)
    post((K - 2) % 4, K + 4)
    post(uK, K + 4, masked=True)
    for u in range(NSLOT):
      wait_scatter(u)

    plsc.subcore_barrier()
    pltpu.sync_copy(acc.at[pl.ds(r0, rows_per_s)],
                    out_hbm.at[c, pl.ds(r0, rows_per_s)])

  return pl.kernel(
      body,
      out_type=jax.ShapeDtypeStruct((NC, N, C), jnp.float32),
      mesh=mesh,
      compiler_params=pltpu.CompilerParams(use_tc_tiling_on_sc=False,
                                           needs_layout_passes=False),
      scratch_types=(
          [pltpu.VMEM((CH,), jnp.int32) for _ in range(NSLOT)]
          + [pltpu.VMEM((CH,), jnp.int32) for _ in range(NSLOT)]
          + [pltpu.VMEM((CH,), jnp.float32) for _ in range(NSLOT)]
          + [pltpu.VMEM((CH, C), jnp.float32) for _ in range(NSLOT)]
          + [pltpu.VMEM_SHARED((N, C), jnp.float32)]
          + [pltpu.SemaphoreType.DMA for _ in range(4 * NSLOT)]
      ),
  )


def _tc_matmul(x, w):
  K, C = w.shape

  def body(x_ref, w_ref, o_ref):
    o_ref[...] = jnp.dot(x_ref[...], w_ref[...],
                         preferred_element_type=jnp.float32)

  return pl.pallas_call(
      body,
      grid=(GRID,),
      in_specs=[pl.BlockSpec((ROW_BLK, K), lambda i: (i, 0)),
                pl.BlockSpec((K, C), lambda i: (0, 0))],
      out_specs=pl.BlockSpec((ROW_BLK, C), lambda i: (i, 0)),
      out_shape=jax.ShapeDtypeStruct((N, C), jnp.float32),
  )(x, w)


def _tc_add_relu_matmul(p, w):
  K, C = w.shape

  def body(p_ref, w_ref, o_ref):
    h = jnp.maximum(p_ref[0] + p_ref[1], 0.0)
    o_ref[...] = jnp.dot(h, w_ref[...], preferred_element_type=jnp.float32)

  return pl.pallas_call(
      body,
      grid=(GRID,),
      in_specs=[pl.BlockSpec((NC, ROW_BLK, K), lambda i: (0, i, 0)),
                pl.BlockSpec((K, C), lambda i: (0, 0))],
      out_specs=pl.BlockSpec((ROW_BLK, C), lambda i: (i, 0)),
      out_shape=jax.ShapeDtypeStruct((N, C), jnp.float32),
  )(p, w)


def _tc_add_softmax2(q2):
  def body(q_ref, o_ref):
    z = q_ref[0] + q_ref[1]

    def sm(zz):
      m = jnp.max(zz, axis=-1, keepdims=True)
      e = jnp.exp(zz - m)
      return e / jnp.sum(e, axis=-1, keepdims=True)

    o_ref[:, pl.ds(0, 64)] = sm(z[:, :64])
    o_ref[:, pl.ds(64, 64)] = sm(z[:, 64:])

  return pl.pallas_call(
      body,
      grid=(GRID,),
      in_specs=[pl.BlockSpec((NC, ROW_BLK // 2, 128), lambda i: (0, i, 0))],
      out_specs=pl.BlockSpec((ROW_BLK // 2, 128), lambda i: (i, 0)),
      out_shape=jax.ShapeDtypeStruct((N // 2, 128), jnp.float32),
  )(q2)


def kernel(x, edge_index, edge_weight, W0, W1):
  E = edge_weight.shape[0]
  h0 = _tc_matmul(x, W0)
  p = _sc_aggregate(CHANNELS, E, 64)(h0, edge_index, edge_weight)
  h1 = _tc_add_relu_matmul(p, W1)
  q = _sc_aggregate(N_LABELS, E, 128)(h1, edge_index, edge_weight)
  q2 = jnp.reshape(q, (NC, N // 2, 2 * N_LABELS))
  return jnp.reshape(_tc_add_softmax2(q2), (N, N_LABELS))
